# Initial kernel scaffold; baseline (speedup 1.0000x reference)
import jax
import jax.numpy as jnp
from jax.experimental import pallas as pl


def kernel(m_ji, e_rbf, a_sbf, kj_idx, ji_idx, W_mkj, b_mkj, W_e, W_a, W_bil):
    raise NotImplementedError("write your pallas kernel here")


# speedup vs baseline: 2.9324x; 2.9324x over previous
import functools

import jax
import jax.numpy as jnp
from jax import lax
from jax.experimental import pallas as pl
from jax.experimental.pallas import tpu as pltpu
from jax.experimental.pallas import tpu_sc as plsc

N_EDGES = 160000
N_ANGLES = 320000
EMBED = 128
N_RBF = 6
N_SPHER_L = 49
N_BIL = 8

_f32 = jnp.float32
_i32 = jnp.int32

_RA = 2000


def _stage_a_body(m_ref, e_ref, wm_ref, b_ref, we_ref, h_ref, g_ref):
    z = jnp.dot(m_ref[...], wm_ref[...], preferred_element_type=_f32) + b_ref[...]
    h_ref[...] = z * jax.nn.sigmoid(z)
    g_ref[...] = jnp.dot(e_ref[...], we_ref[...], preferred_element_type=_f32)


def _stage_a(m_ji, e_rbf, W_mkj, b_mkj, W_e):
    return pl.pallas_call(
        _stage_a_body,
        grid=(N_EDGES // _RA,),
        in_specs=[
            pl.BlockSpec((_RA, EMBED), lambda i: (i, 0)),
            pl.BlockSpec((_RA, N_RBF), lambda i: (i, 0)),
            pl.BlockSpec((EMBED, EMBED), lambda i: (0, 0)),
            pl.BlockSpec((1, EMBED), lambda i: (0, 0)),
            pl.BlockSpec((N_RBF, EMBED), lambda i: (0, 0)),
        ],
        out_specs=[pl.BlockSpec((_RA, EMBED), lambda i: (i, 0))] * 2,
        out_shape=[jax.ShapeDtypeStruct((N_EDGES, EMBED), _f32)] * 2,
    )(m_ji, e_rbf, W_mkj, b_mkj.reshape(1, EMBED), W_e)


_RB = 1000


def _stage_b_body(hk_ref, gj_ref, a_ref, wa_ref, wcat_ref, out_ref):
    x = hk_ref[...] * gj_ref[...]
    a = jnp.dot(a_ref[...], wa_ref[...], preferred_element_type=_f32)
    xa = jnp.concatenate(
        [(x * a[:, j : j + 1]).astype(jnp.bfloat16) for j in range(N_BIL)], axis=1
    )
    out_ref[...] = jnp.dot(xa, wcat_ref[...], preferred_element_type=_f32)


def _stage_b(hk, gj, a_sbf, W_a, W_cat):
    return pl.pallas_call(
        _stage_b_body,
        grid=(N_ANGLES // _RB,),
        in_specs=[
            pl.BlockSpec((_RB, EMBED), lambda i: (i, 0)),
            pl.BlockSpec((_RB, EMBED), lambda i: (i, 0)),
            pl.BlockSpec((_RB, N_SPHER_L), lambda i: (i, 0)),
            pl.BlockSpec((N_SPHER_L, N_BIL), lambda i: (0, 0)),
            pl.BlockSpec((N_BIL * EMBED, EMBED), lambda i: (0, 0)),
        ],
        out_specs=pl.BlockSpec((_RB, EMBED), lambda i: (i, 0)),
        out_shape=jax.ShapeDtypeStruct((N_ANGLES, EMBED), _f32),
    )(hk, gj, a_sbf, W_a, W_cat)


_NC = 2
_NS = 16
_NW = _NC * _NS
_WPT = N_ANGLES // _NW
_CH = 128
_NFULL = _WPT // _CH
_TAIL = _WPT - _NFULL * _CH


def _sc_gather(h, g, kj_idx, ji_idx):
    mesh = plsc.VectorSubcoreMesh(core_axis_name="c", subcore_axis_name="s")

    @functools.partial(
        pl.kernel,
        out_type=[jax.ShapeDtypeStruct((N_ANGLES, EMBED), _f32)] * 2,
        mesh=mesh,
        compiler_params=pltpu.CompilerParams(needs_layout_passes=False),
        scratch_types=[
            pltpu.VMEM((_WPT,), _i32),
            pltpu.VMEM((_WPT,), _i32),
            pltpu.VMEM((_CH, EMBED), _f32),
            pltpu.VMEM((_CH, EMBED), _f32),
            pltpu.SemaphoreType.DMA,
            pltpu.SemaphoreType.DMA,
        ],
    )
    def k(h_hbm, g_hbm, kj_hbm, ji_hbm, hk_hbm, gj_hbm, kjv, jiv, hbuf, gbuf, s1, s2):
        wid = lax.axis_index("s") * _NC + lax.axis_index("c")
        base = pl.multiple_of(wid * _WPT, 8)
        pltpu.sync_copy(kj_hbm.at[pl.ds(base, _WPT)], kjv)
        pltpu.sync_copy(ji_hbm.at[pl.ds(base, _WPT)], jiv)

        def body(kk, carry):
            off = pl.multiple_of(kk * _CH, 8)
            c1 = pltpu.async_copy(h_hbm.at[kjv.at[pl.ds(off, _CH)]], hbuf, s1)
            c2 = pltpu.async_copy(g_hbm.at[jiv.at[pl.ds(off, _CH)]], gbuf, s2)
            c1.wait()
            c2.wait()
            o1 = pltpu.async_copy(hbuf, hk_hbm.at[pl.ds(base + off, _CH)], s1)
            o2 = pltpu.async_copy(gbuf, gj_hbm.at[pl.ds(base + off, _CH)], s2)
            o1.wait()
            o2.wait()
            return carry

        lax.fori_loop(0, _NFULL, body, jnp.int32(0))
        toff = _NFULL * _CH
        c1 = pltpu.async_copy(h_hbm.at[kjv.at[pl.ds(toff, _TAIL)]],
                              hbuf.at[pl.ds(0, _TAIL)], s1)
        c2 = pltpu.async_copy(g_hbm.at[jiv.at[pl.ds(toff, _TAIL)]],
                              gbuf.at[pl.ds(0, _TAIL)], s2)
        c1.wait()
        c2.wait()
        o1 = pltpu.async_copy(hbuf.at[pl.ds(0, _TAIL)],
                              hk_hbm.at[pl.ds(base + toff, _TAIL)], s1)
        o2 = pltpu.async_copy(gbuf.at[pl.ds(0, _TAIL)],
                              gj_hbm.at[pl.ds(base + toff, _TAIL)], s2)
        o1.wait()
        o2.wait()

    return k(h, g, kj_idx, ji_idx)


_BIN = 10000
_NPASS = 8
_TRASH = 10000
_ACC_ROWS = 10112
_WPS = N_ANGLES // _NS
_APL = _WPS // 16
_LCAPL = 128


def _sc_scatter(aggr, ji_idx):
    mesh = plsc.VectorSubcoreMesh(core_axis_name="c", subcore_axis_name="s")

    @functools.partial(
        pl.kernel,
        out_type=jax.ShapeDtypeStruct((N_EDGES, EMBED), _f32),
        mesh=mesh,
        compiler_params=pltpu.CompilerParams(needs_layout_passes=False),
        scratch_types=[
            pltpu.VMEM((_WPS,), _i32),
            pltpu.VMEM((16, _LCAPL), _i32),
            pltpu.VMEM((16, _LCAPL), _i32),
            pltpu.VMEM((_CH, EMBED), _f32),
            pltpu.VMEM_SHARED((_ACC_ROWS, EMBED), _f32),
            pltpu.SemaphoreType.DMA,
        ],
    )
    def k(aggr_hbm, ji_hbm, out_hbm, jiv, wlist, tgt, rows, acc, sem):
        cid = lax.axis_index("c")
        sid = lax.axis_index("s")
        base = pl.multiple_of(sid * _WPS, 8)
        pltpu.sync_copy(ji_hbm.at[pl.ds(base, _WPS)], jiv)

        zv = jnp.zeros((16,), _f32)
        lanes = lax.iota(_i32, 16)
        ones16 = jnp.ones((16,), _i32)
        zeros16 = jnp.zeros((16,), _i32)
        trash16 = jnp.full((16,), _TRASH, _i32)
        lane_starts = lanes * _APL
        wdefault = zeros16 + base

        def zrow(r, carry):
            for c8 in range(EMBED // 16):
                rows[r, pl.ds(c8 * 16, 16)] = zv
            return carry

        def lrow(r, carry):
            for c8 in range(_LCAPL // 16):
                wlist[r, pl.ds(c8 * 16, 16)] = wdefault
            return carry

        lax.fori_loop(0, 16, lrow, jnp.int32(0))

        for p in range(_NPASS):
            lo = pl.multiple_of((2 * p + cid) * _BIN, 8)
            lax.fori_loop(0, _CH, zrow, jnp.int32(0))
            zoff = pl.multiple_of(sid * 632, 8)
            for z in range(5):
                sz = 128 if z < 4 else 120
                pltpu.sync_copy(rows.at[pl.ds(0, sz)],
                                acc.at[pl.ds(pl.multiple_of(zoff + z * 128, 8), sz)])
            def trow(r, carry):
                for c8 in range(_LCAPL // 16):
                    tgt[r, pl.ds(c8 * 16, 16)] = trash16
                return carry

            lax.fori_loop(0, 16, trow, jnp.int32(0))
            plsc.subcore_barrier()

            def cgroup(i, cnt):
                idx = lane_starts + i
                jvec = plsc.load_gather(jiv, [idx])
                inb = (jvec >= lo) & (jvec < lo + _BIN)
                pos = jnp.minimum(cnt, _LCAPL - 1)
                plsc.store_scatter(wlist, [lanes, pos], base + idx, mask=inb)
                plsc.store_scatter(tgt, [lanes, pos], jvec - lo, mask=inb)
                return cnt + jnp.where(inb, ones16, zeros16)

            lax.fori_loop(0, _APL, cgroup, zeros16)

            for ln in range(16):
                pltpu.async_copy(aggr_hbm.at[wlist.at[ln]], rows, sem).wait()
                pltpu.sync_copy(rows, acc.at[tgt.at[ln]], add=True)
            plsc.subcore_barrier()

            woff = pl.multiple_of(sid * 624, 8)

            def wchunk(z, sz):
                aoff = pl.multiple_of(woff + z * 128, 8)
                pltpu.sync_copy(acc.at[pl.ds(aoff, sz)], rows.at[pl.ds(0, sz)])
                pltpu.sync_copy(rows.at[pl.ds(0, sz)],
                                out_hbm.at[pl.ds(pl.multiple_of(lo + aoff, 8), sz)])

            for z in range(4):
                wchunk(z, 128)

            @pl.when(sid < 15)
            def _():
                wchunk(4, 112)

            @pl.when(sid == 15)
            def _():
                wchunk(4, 128)

            plsc.subcore_barrier()

    return k(aggr, ji_idx)


def kernel(m_ji, e_rbf, a_sbf, kj_idx, ji_idx, W_mkj, b_mkj, W_e, W_a, W_bil):
    kj_idx = kj_idx.astype(_i32)
    ji_idx = ji_idx.astype(_i32)
    h, g = _stage_a(m_ji, e_rbf, W_mkj, b_mkj, W_e)
    hk, gj = _sc_gather(h, g, kj_idx, ji_idx)
    W_cat = jnp.transpose(W_bil, (1, 2, 0)).reshape(N_BIL * EMBED, EMBED)
    W_cat = W_cat.astype(jnp.bfloat16)
    aggr = _stage_b(hk, gj, a_sbf, W_a, W_cat)
    return _sc_scatter(aggr, ji_idx)


# speedup vs baseline: 3.0202x; 1.0300x over previous
import functools

import jax
import jax.numpy as jnp
from jax import lax
from jax.experimental import pallas as pl
from jax.experimental.pallas import tpu as pltpu
from jax.experimental.pallas import tpu_sc as plsc

N_EDGES = 160000
N_ANGLES = 320000
EMBED = 128
N_RBF = 6
N_SPHER_L = 49
N_BIL = 8

_f32 = jnp.float32
_bf16 = jnp.bfloat16
_i32 = jnp.int32

_RA = 2000


def _stage_a_body(m_ref, e_ref, wm_ref, b_ref, we_ref, h_ref, g_ref):
    z = jnp.dot(m_ref[...], wm_ref[...], preferred_element_type=_f32) + b_ref[...]
    h_ref[...] = z * jax.nn.sigmoid(z)
    g_ref[...] = jnp.dot(e_ref[...], we_ref[...], preferred_element_type=_f32)


def _stage_a(m_ji, e_rbf, W_mkj, b_mkj, W_e):
    return pl.pallas_call(
        _stage_a_body,
        grid=(N_EDGES // _RA,),
        in_specs=[
            pl.BlockSpec((_RA, EMBED), lambda i: (i, 0)),
            pl.BlockSpec((_RA, N_RBF), lambda i: (i, 0)),
            pl.BlockSpec((EMBED, EMBED), lambda i: (0, 0)),
            pl.BlockSpec((1, EMBED), lambda i: (0, 0)),
            pl.BlockSpec((N_RBF, EMBED), lambda i: (0, 0)),
        ],
        out_specs=[pl.BlockSpec((_RA, EMBED), lambda i: (i, 0))] * 2,
        out_shape=[jax.ShapeDtypeStruct((N_EDGES, EMBED), _f32)] * 2,
    )(m_ji, e_rbf, W_mkj, b_mkj.reshape(1, EMBED), W_e)


_RB = 1000


def _stage_b_body(hk_ref, gj_ref, a_ref, wa_ref, wcat_ref, out_ref):
    x = hk_ref[...] * gj_ref[...]
    a = jnp.dot(a_ref[...], wa_ref[...], preferred_element_type=_f32)
    xa = jnp.concatenate(
        [(x * a[:, j : j + 1]).astype(_bf16) for j in range(N_BIL)], axis=1
    )
    out_ref[...] = jnp.dot(xa, wcat_ref[...], preferred_element_type=_f32)


def _stage_b(hk, gj, a_sbf, W_a, W_cat):
    return pl.pallas_call(
        _stage_b_body,
        grid=(N_ANGLES // _RB,),
        in_specs=[
            pl.BlockSpec((_RB, EMBED), lambda i: (i, 0)),
            pl.BlockSpec((_RB, EMBED), lambda i: (i, 0)),
            pl.BlockSpec((_RB, N_SPHER_L), lambda i: (i, 0)),
            pl.BlockSpec((N_SPHER_L, N_BIL), lambda i: (0, 0)),
            pl.BlockSpec((N_BIL * EMBED, EMBED), lambda i: (0, 0)),
        ],
        out_specs=pl.BlockSpec((_RB, EMBED), lambda i: (i, 0)),
        out_shape=jax.ShapeDtypeStruct((N_ANGLES, EMBED), _f32),
    )(hk, gj, a_sbf, W_a, W_cat)


_NC = 2
_NS = 16
_NW = _NC * _NS
_WPT = N_ANGLES // _NW
_CH = 128
_NFULL = _WPT // _CH
_TAIL = _WPT - _NFULL * _CH
_GB = 3


def _sc_gather(h, g, kj_idx, ji_idx):
    mesh = plsc.VectorSubcoreMesh(core_axis_name="c", subcore_axis_name="s")

    @functools.partial(
        pl.kernel,
        out_type=[jax.ShapeDtypeStruct((N_ANGLES, EMBED), _f32)] * 2,
        mesh=mesh,
        compiler_params=pltpu.CompilerParams(needs_layout_passes=False),
        scratch_types=[
            pltpu.VMEM((_WPT,), _i32),
            pltpu.VMEM((_WPT,), _i32),
        ]
        + [pltpu.VMEM((_CH, EMBED), _f32)] * (2 * _GB)
        + [pltpu.SemaphoreType.DMA] * (4 * _GB),
    )
    def k(h_hbm, g_hbm, kj_hbm, ji_hbm, hk_hbm, gj_hbm, kjv, jiv, *rest):
        hbufs = rest[0:_GB]
        gbufs = rest[_GB:2 * _GB]
        sgh = rest[2 * _GB:3 * _GB]
        sgg = rest[3 * _GB:4 * _GB]
        swh = rest[4 * _GB:5 * _GB]
        swg = rest[5 * _GB:6 * _GB]
        wid = lax.axis_index("s") * _NC + lax.axis_index("c")
        base = pl.multiple_of(wid * _WPT, 8)
        pltpu.sync_copy(kj_hbm.at[pl.ds(base, _WPT)], kjv)
        pltpu.sync_copy(ji_hbm.at[pl.ds(base, _WPT)], jiv)

        for b in range(_GB):
            off = b * _CH
            pltpu.async_copy(h_hbm.at[kjv.at[pl.ds(off, _CH)]], hbufs[b], sgh[b])
            pltpu.async_copy(g_hbm.at[jiv.at[pl.ds(off, _CH)]], gbufs[b], sgg[b])

        def body(m, carry):
            for b in range(_GB):
                off = pl.multiple_of((m * _GB + b) * _CH, 8)
                pltpu.make_async_copy(h_hbm.at[kjv.at[pl.ds(off, _CH)]],
                                      hbufs[b], sgh[b]).wait()
                pltpu.async_copy(hbufs[b], hk_hbm.at[pl.ds(base + off, _CH)],
                                 swh[b])
                pltpu.make_async_copy(g_hbm.at[jiv.at[pl.ds(off, _CH)]],
                                      gbufs[b], sgg[b]).wait()
                pltpu.async_copy(gbufs[b], gj_hbm.at[pl.ds(base + off, _CH)],
                                 swg[b])

            @pl.when(m < _NFULL // _GB - 1)
            def _():
                for b in range(_GB):
                    noff = pl.multiple_of((m * _GB + _GB + b) * _CH, 8)
                    pltpu.make_async_copy(
                        hbufs[b], hk_hbm.at[pl.ds(base, _CH)], swh[b]).wait()
                    pltpu.async_copy(h_hbm.at[kjv.at[pl.ds(noff, _CH)]],
                                     hbufs[b], sgh[b])
                    pltpu.make_async_copy(
                        gbufs[b], gj_hbm.at[pl.ds(base, _CH)], swg[b]).wait()
                    pltpu.async_copy(g_hbm.at[jiv.at[pl.ds(noff, _CH)]],
                                     gbufs[b], sgg[b])

            return carry

        lax.fori_loop(0, _NFULL // _GB, body, jnp.int32(0))
        for b in range(_GB):
            pltpu.make_async_copy(hbufs[b], hk_hbm.at[pl.ds(base, _CH)],
                                  swh[b]).wait()
            pltpu.make_async_copy(gbufs[b], gj_hbm.at[pl.ds(base, _CH)],
                                  swg[b]).wait()
        toff = _NFULL * _CH
        c1 = pltpu.async_copy(h_hbm.at[kjv.at[pl.ds(toff, _TAIL)]],
                              hbufs[0].at[pl.ds(0, _TAIL)], sgh[0])
        c2 = pltpu.async_copy(g_hbm.at[jiv.at[pl.ds(toff, _TAIL)]],
                              gbufs[0].at[pl.ds(0, _TAIL)], sgg[0])
        c1.wait()
        c2.wait()
        o1 = pltpu.async_copy(hbufs[0].at[pl.ds(0, _TAIL)],
                              hk_hbm.at[pl.ds(base + toff, _TAIL)], sgh[0])
        o2 = pltpu.async_copy(gbufs[0].at[pl.ds(0, _TAIL)],
                              gj_hbm.at[pl.ds(base + toff, _TAIL)], sgg[0])
        o1.wait()
        o2.wait()

    return k(h, g, kj_idx, ji_idx)


_BIN = 10000
_NPASS = 8
_TRASH = 10000
_ACC_ROWS = 10112
_WPS = N_ANGLES // _NS
_APL = _WPS // 16
_SCH = 64
_NREG = 32


def _sc_scatter(aggr, ji_idx):
    mesh = plsc.VectorSubcoreMesh(core_axis_name="c", subcore_axis_name="s")

    @functools.partial(
        pl.kernel,
        out_type=jax.ShapeDtypeStruct((N_EDGES, EMBED), _f32),
        mesh=mesh,
        compiler_params=pltpu.CompilerParams(needs_layout_passes=False),
        scratch_types=[
            pltpu.VMEM((_WPS,), _i32),
            pltpu.VMEM((_NREG, _SCH), _i32),
            pltpu.VMEM((_NREG, _SCH), _i32),
            pltpu.VMEM((_SCH, EMBED), _f32),
            pltpu.VMEM((_SCH, EMBED), _f32),
            pltpu.VMEM_SHARED((_ACC_ROWS, EMBED), _f32),
            pltpu.SemaphoreType.DMA,
            pltpu.SemaphoreType.DMA,
            pltpu.SemaphoreType.DMA,
            pltpu.SemaphoreType.DMA,
        ],
    )
    def k(aggr_hbm, ji_hbm, out_hbm, jiv, wlist, tgt, rows0, rows1, acc,
          sg0, sg1, ss0, ss1):
        rows = (rows0, rows1)
        sg = (sg0, sg1)
        ss = (ss0, ss1)
        cid = lax.axis_index("c")
        sid = lax.axis_index("s")
        base = pl.multiple_of(sid * _WPS, 8)
        pltpu.sync_copy(ji_hbm.at[pl.ds(base, _WPS)], jiv)

        zv = jnp.zeros((16,), _f32)
        lanes = lax.iota(_i32, 16)
        ones16 = jnp.ones((16,), _i32)
        zeros16 = jnp.zeros((16,), _i32)
        trash16 = jnp.full((16,), _TRASH, _i32)
        lane_starts = lanes * _APL
        lane_rows = lanes * 2
        wdefault = zeros16 + base

        def zrow(r, carry):
            for c in range(EMBED // 16):
                rows0[r, pl.ds(c * 16, 16)] = zv
            return carry

        def lrow(r, carry):
            for c8 in range(_SCH // 16):
                wlist[r, pl.ds(c8 * 16, 16)] = wdefault
            return carry

        lax.fori_loop(0, _NREG, lrow, jnp.int32(0))

        for p in range(_NPASS):
            lo = pl.multiple_of((2 * p + cid) * _BIN, 8)
            lax.fori_loop(0, _SCH, zrow, jnp.int32(0))
            zoff = pl.multiple_of(sid * 632, 8)
            for z in range(10):
                sz = 64 if z < 9 else 56
                pltpu.sync_copy(rows0.at[pl.ds(0, sz)],
                                acc.at[pl.ds(pl.multiple_of(zoff + z * 64, 8), sz)])

            def trow(r, carry):
                for c8 in range(_SCH // 16):
                    tgt[r, pl.ds(c8 * 16, 16)] = trash16
                return carry

            lax.fori_loop(0, _NREG, trow, jnp.int32(0))
            plsc.subcore_barrier()

            def cgroup(i, cnt):
                idx = lane_starts + i
                jvec = plsc.load_gather(jiv, [idx])
                inb = (jvec >= lo) & (jvec < lo + _BIN)
                cc = jnp.minimum(cnt, 2 * _SCH - 1)
                prow = lane_rows + (cc >> 6)
                pcol = cc & (_SCH - 1)
                plsc.store_scatter(wlist, [prow, pcol], base + idx, mask=inb)
                plsc.store_scatter(tgt, [prow, pcol], jvec - lo, mask=inb)
                return cnt + jnp.where(inb, ones16, zeros16)

            lax.fori_loop(0, _APL, cgroup, zeros16)

            pltpu.async_copy(aggr_hbm.at[wlist.at[0]], rows[0], sg[0])
            for r in range(_NREG):
                b = r % 2
                if r + 1 < _NREG:
                    b2 = (r + 1) % 2
                    if r >= 1:
                        pltpu.make_async_copy(rows[b2], acc.at[tgt.at[r - 1]],
                                              ss[b2]).wait()
                    pltpu.async_copy(aggr_hbm.at[wlist.at[r + 1]], rows[b2],
                                     sg[b2])
                pltpu.make_async_copy(aggr_hbm.at[wlist.at[r]], rows[b],
                                      sg[b]).wait()
                pltpu.async_copy(rows[b], acc.at[tgt.at[r]], ss[b], add=True)
            pltpu.make_async_copy(rows[0], acc.at[tgt.at[_NREG - 2]],
                                  ss[0]).wait()
            pltpu.make_async_copy(rows[1], acc.at[tgt.at[_NREG - 1]],
                                  ss[1]).wait()
            plsc.subcore_barrier()

            woff = pl.multiple_of(sid * 624, 8)

            def wchunk(z, sz):
                aoff = pl.multiple_of(woff + z * 64, 8)
                pltpu.sync_copy(acc.at[pl.ds(aoff, sz)], rows0.at[pl.ds(0, sz)])
                pltpu.sync_copy(rows0.at[pl.ds(0, sz)],
                                out_hbm.at[pl.ds(pl.multiple_of(lo + aoff, 8), sz)])

            for z in range(9):
                wchunk(z, 64)

            @pl.when(sid < 15)
            def _():
                wchunk(9, 48)

            @pl.when(sid == 15)
            def _():
                wchunk(9, 64)

            plsc.subcore_barrier()

    return k(aggr, ji_idx)


def kernel(m_ji, e_rbf, a_sbf, kj_idx, ji_idx, W_mkj, b_mkj, W_e, W_a, W_bil):
    kj_idx = kj_idx.astype(_i32)
    ji_idx = ji_idx.astype(_i32)
    h, g = _stage_a(m_ji, e_rbf, W_mkj, b_mkj, W_e)
    hk, gj = _sc_gather(h, g, kj_idx, ji_idx)
    W_cat = jnp.transpose(W_bil, (1, 2, 0)).reshape(N_BIL * EMBED, EMBED)
    W_cat = W_cat.astype(jnp.bfloat16)
    aggr = _stage_b(hk, gj, a_sbf, W_a, W_cat)
    return _sc_scatter(aggr, ji_idx)


# speedup vs baseline: 3.1139x; 1.0310x over previous
import functools

import jax
import jax.numpy as jnp
from jax import lax
from jax.experimental import pallas as pl
from jax.experimental.pallas import tpu as pltpu
from jax.experimental.pallas import tpu_sc as plsc

N_EDGES = 160000
N_ANGLES = 320000
EMBED = 128
N_RBF = 6
N_SPHER_L = 49
N_BIL = 8

_f32 = jnp.float32
_bf16 = jnp.bfloat16
_i32 = jnp.int32

_RA = 2000


def _stage_a_body(m_ref, e_ref, wm_ref, b_ref, we_ref, h_ref, g_ref):
    z = jnp.dot(m_ref[...], wm_ref[...], preferred_element_type=_f32) + b_ref[...]
    h_ref[...] = z * jax.nn.sigmoid(z)
    g_ref[...] = jnp.dot(e_ref[...], we_ref[...], preferred_element_type=_f32)


def _stage_a(m_ji, e_rbf, W_mkj, b_mkj, W_e):
    return pl.pallas_call(
        _stage_a_body,
        grid=(N_EDGES // _RA,),
        in_specs=[
            pl.BlockSpec((_RA, EMBED), lambda i: (i, 0)),
            pl.BlockSpec((_RA, N_RBF), lambda i: (i, 0)),
            pl.BlockSpec((EMBED, EMBED), lambda i: (0, 0)),
            pl.BlockSpec((1, EMBED), lambda i: (0, 0)),
            pl.BlockSpec((N_RBF, EMBED), lambda i: (0, 0)),
        ],
        out_specs=[pl.BlockSpec((_RA, EMBED), lambda i: (i, 0))] * 2,
        out_shape=[jax.ShapeDtypeStruct((N_EDGES, EMBED), _f32)] * 2,
    )(m_ji, e_rbf, W_mkj, b_mkj.reshape(1, EMBED), W_e)


_RB = 1000


def _stage_b_body(hk_ref, gj_ref, a_ref, wa_ref, wall_ref, e_ref, out_ref):
    x = (hk_ref[...] * gj_ref[...]).astype(_bf16)
    a = jnp.dot(a_ref[...], wa_ref[...],
                preferred_element_type=_f32).astype(_bf16)
    y = jnp.dot(x, wall_ref[...], preferred_element_type=_f32)
    abig = jnp.dot(a, e_ref[...], preferred_element_type=_f32)
    xx = abig * y
    acc = xx[:, 0:EMBED]
    for j in range(1, N_BIL):
        acc = acc + xx[:, j * EMBED : (j + 1) * EMBED]
    out_ref[...] = acc


def _stage_b(hk, gj, a_sbf, W_a, W_all, E_exp):
    return pl.pallas_call(
        _stage_b_body,
        grid=(N_ANGLES // _RB,),
        in_specs=[
            pl.BlockSpec((_RB, EMBED), lambda i: (i, 0)),
            pl.BlockSpec((_RB, EMBED), lambda i: (i, 0)),
            pl.BlockSpec((_RB, N_SPHER_L), lambda i: (i, 0)),
            pl.BlockSpec((N_SPHER_L, N_BIL), lambda i: (0, 0)),
            pl.BlockSpec((EMBED, N_BIL * EMBED), lambda i: (0, 0)),
            pl.BlockSpec((N_BIL, N_BIL * EMBED), lambda i: (0, 0)),
        ],
        out_specs=pl.BlockSpec((_RB, EMBED), lambda i: (i, 0)),
        out_shape=jax.ShapeDtypeStruct((N_ANGLES, EMBED), _f32),
    )(hk, gj, a_sbf, W_a, W_all, E_exp)


_NC = 2
_NS = 16
_NW = _NC * _NS
_WPT = N_ANGLES // _NW
_CH = 128
_NFULL = _WPT // _CH
_TAIL = _WPT - _NFULL * _CH
_GB = 3


def _sc_gather(h, g, kj_idx, ji_idx):
    mesh = plsc.VectorSubcoreMesh(core_axis_name="c", subcore_axis_name="s")

    @functools.partial(
        pl.kernel,
        out_type=[jax.ShapeDtypeStruct((N_ANGLES, EMBED), _f32)] * 2,
        mesh=mesh,
        compiler_params=pltpu.CompilerParams(needs_layout_passes=False),
        scratch_types=[
            pltpu.VMEM((_WPT,), _i32),
            pltpu.VMEM((_WPT,), _i32),
        ]
        + [pltpu.VMEM((_CH, EMBED), _f32)] * (2 * _GB)
        + [pltpu.SemaphoreType.DMA] * (4 * _GB),
    )
    def k(h_hbm, g_hbm, kj_hbm, ji_hbm, hk_hbm, gj_hbm, kjv, jiv, *rest):
        hbufs = rest[0:_GB]
        gbufs = rest[_GB:2 * _GB]
        sgh = rest[2 * _GB:3 * _GB]
        sgg = rest[3 * _GB:4 * _GB]
        swh = rest[4 * _GB:5 * _GB]
        swg = rest[5 * _GB:6 * _GB]
        wid = lax.axis_index("s") * _NC + lax.axis_index("c")
        base = pl.multiple_of(wid * _WPT, 8)
        pltpu.sync_copy(kj_hbm.at[pl.ds(base, _WPT)], kjv)
        pltpu.sync_copy(ji_hbm.at[pl.ds(base, _WPT)], jiv)

        for b in range(_GB):
            off = b * _CH
            pltpu.async_copy(h_hbm.at[kjv.at[pl.ds(off, _CH)]], hbufs[b], sgh[b])
            pltpu.async_copy(g_hbm.at[jiv.at[pl.ds(off, _CH)]], gbufs[b], sgg[b])

        def body(m, carry):
            for b in range(_GB):
                off = pl.multiple_of((m * _GB + b) * _CH, 8)
                pltpu.make_async_copy(h_hbm.at[kjv.at[pl.ds(off, _CH)]],
                                      hbufs[b], sgh[b]).wait()
                pltpu.async_copy(hbufs[b], hk_hbm.at[pl.ds(base + off, _CH)],
                                 swh[b])
                pltpu.make_async_copy(g_hbm.at[jiv.at[pl.ds(off, _CH)]],
                                      gbufs[b], sgg[b]).wait()
                pltpu.async_copy(gbufs[b], gj_hbm.at[pl.ds(base + off, _CH)],
                                 swg[b])

            @pl.when(m < _NFULL // _GB - 1)
            def _():
                for b in range(_GB):
                    noff = pl.multiple_of((m * _GB + _GB + b) * _CH, 8)
                    pltpu.make_async_copy(
                        hbufs[b], hk_hbm.at[pl.ds(base, _CH)], swh[b]).wait()
                    pltpu.async_copy(h_hbm.at[kjv.at[pl.ds(noff, _CH)]],
                                     hbufs[b], sgh[b])
                    pltpu.make_async_copy(
                        gbufs[b], gj_hbm.at[pl.ds(base, _CH)], swg[b]).wait()
                    pltpu.async_copy(g_hbm.at[jiv.at[pl.ds(noff, _CH)]],
                                     gbufs[b], sgg[b])

            return carry

        lax.fori_loop(0, _NFULL // _GB, body, jnp.int32(0))
        for b in range(_GB):
            pltpu.make_async_copy(hbufs[b], hk_hbm.at[pl.ds(base, _CH)],
                                  swh[b]).wait()
            pltpu.make_async_copy(gbufs[b], gj_hbm.at[pl.ds(base, _CH)],
                                  swg[b]).wait()
        toff = _NFULL * _CH
        c1 = pltpu.async_copy(h_hbm.at[kjv.at[pl.ds(toff, _TAIL)]],
                              hbufs[0].at[pl.ds(0, _TAIL)], sgh[0])
        c2 = pltpu.async_copy(g_hbm.at[jiv.at[pl.ds(toff, _TAIL)]],
                              gbufs[0].at[pl.ds(0, _TAIL)], sgg[0])
        c1.wait()
        c2.wait()
        o1 = pltpu.async_copy(hbufs[0].at[pl.ds(0, _TAIL)],
                              hk_hbm.at[pl.ds(base + toff, _TAIL)], sgh[0])
        o2 = pltpu.async_copy(gbufs[0].at[pl.ds(0, _TAIL)],
                              gj_hbm.at[pl.ds(base + toff, _TAIL)], sgg[0])
        o1.wait()
        o2.wait()

    return k(h, g, kj_idx, ji_idx)


_BIN = 10000
_NPASS = 8
_TRASH = 10000
_ACC_ROWS = 10112
_WPS = N_ANGLES // _NS
_APL = _WPS // 16
_SCH = 64
_NREG = 32


def _sc_scatter(aggr, ji_idx):
    mesh = plsc.VectorSubcoreMesh(core_axis_name="c", subcore_axis_name="s")

    @functools.partial(
        pl.kernel,
        out_type=jax.ShapeDtypeStruct((N_EDGES, EMBED), _f32),
        mesh=mesh,
        compiler_params=pltpu.CompilerParams(needs_layout_passes=False),
        scratch_types=[
            pltpu.VMEM((_WPS,), _i32),
            pltpu.VMEM((_NREG, _SCH), _i32),
            pltpu.VMEM((_NREG, _SCH), _i32),
            pltpu.VMEM((_SCH, EMBED), _f32),
            pltpu.VMEM((_SCH, EMBED), _f32),
            pltpu.VMEM_SHARED((_ACC_ROWS, EMBED), _f32),
            pltpu.SemaphoreType.DMA,
            pltpu.SemaphoreType.DMA,
            pltpu.SemaphoreType.DMA,
            pltpu.SemaphoreType.DMA,
        ],
    )
    def k(aggr_hbm, ji_hbm, out_hbm, jiv, wlist, tgt, rows0, rows1, acc,
          sg0, sg1, ss0, ss1):
        rows = (rows0, rows1)
        sg = (sg0, sg1)
        ss = (ss0, ss1)
        cid = lax.axis_index("c")
        sid = lax.axis_index("s")
        base = pl.multiple_of(sid * _WPS, 8)
        pltpu.sync_copy(ji_hbm.at[pl.ds(base, _WPS)], jiv)

        zv = jnp.zeros((16,), _f32)
        lanes = lax.iota(_i32, 16)
        ones16 = jnp.ones((16,), _i32)
        zeros16 = jnp.zeros((16,), _i32)
        trash16 = jnp.full((16,), _TRASH, _i32)
        lane_starts = lanes * _APL
        lane_rows = lanes * 2
        wdefault = zeros16 + base

        def zrow(r, carry):
            for c in range(EMBED // 16):
                rows0[r, pl.ds(c * 16, 16)] = zv
            return carry

        def lrow(r, carry):
            for c8 in range(_SCH // 16):
                wlist[r, pl.ds(c8 * 16, 16)] = wdefault
            return carry

        lax.fori_loop(0, _NREG, lrow, jnp.int32(0))

        for p in range(_NPASS):
            lo = pl.multiple_of((2 * p + cid) * _BIN, 8)
            lax.fori_loop(0, _SCH, zrow, jnp.int32(0))
            zoff = pl.multiple_of(sid * 632, 8)
            for z in range(10):
                sz = 64 if z < 9 else 56
                pltpu.sync_copy(rows0.at[pl.ds(0, sz)],
                                acc.at[pl.ds(pl.multiple_of(zoff + z * 64, 8), sz)])

            def trow(r, carry):
                tr = zeros16 + (_TRASH + lax.rem(sid * _NREG + r, 112))
                for c8 in range(_SCH // 16):
                    tgt[r, pl.ds(c8 * 16, 16)] = tr
                return carry

            lax.fori_loop(0, _NREG, trow, jnp.int32(0))
            plsc.subcore_barrier()

            def cgroup(i, cnt):
                idx = lane_starts + i
                jvec = plsc.load_gather(jiv, [idx])
                inb = (jvec >= lo) & (jvec < lo + _BIN)
                cc = jnp.minimum(cnt, 2 * _SCH - 1)
                prow = lane_rows + (cc >> 6)
                pcol = cc & (_SCH - 1)
                plsc.store_scatter(wlist, [prow, pcol], base + idx, mask=inb)
                plsc.store_scatter(tgt, [prow, pcol], jvec - lo, mask=inb)
                return cnt + jnp.where(inb, ones16, zeros16)

            lax.fori_loop(0, _APL, cgroup, zeros16)

            pltpu.async_copy(aggr_hbm.at[wlist.at[0]], rows[0], sg[0])
            for r in range(_NREG):
                b = r % 2
                if r + 1 < _NREG:
                    b2 = (r + 1) % 2
                    if r >= 1:
                        pltpu.make_async_copy(rows[b2], acc.at[tgt.at[r - 1]],
                                              ss[b2]).wait()
                    pltpu.async_copy(aggr_hbm.at[wlist.at[r + 1]], rows[b2],
                                     sg[b2])
                pltpu.make_async_copy(aggr_hbm.at[wlist.at[r]], rows[b],
                                      sg[b]).wait()
                pltpu.async_copy(rows[b], acc.at[tgt.at[r]], ss[b], add=True)
            pltpu.make_async_copy(rows[0], acc.at[tgt.at[_NREG - 2]],
                                  ss[0]).wait()
            pltpu.make_async_copy(rows[1], acc.at[tgt.at[_NREG - 1]],
                                  ss[1]).wait()
            plsc.subcore_barrier()

            woff = pl.multiple_of(sid * 624, 8)

            def wchunk(z, sz):
                aoff = pl.multiple_of(woff + z * 64, 8)
                pltpu.sync_copy(acc.at[pl.ds(aoff, sz)], rows0.at[pl.ds(0, sz)])
                pltpu.sync_copy(rows0.at[pl.ds(0, sz)],
                                out_hbm.at[pl.ds(pl.multiple_of(lo + aoff, 8), sz)])

            for z in range(9):
                wchunk(z, 64)

            @pl.when(sid < 15)
            def _():
                wchunk(9, 48)

            @pl.when(sid == 15)
            def _():
                wchunk(9, 64)

            plsc.subcore_barrier()

    return k(aggr, ji_idx)


def kernel(m_ji, e_rbf, a_sbf, kj_idx, ji_idx, W_mkj, b_mkj, W_e, W_a, W_bil):
    kj_idx = kj_idx.astype(_i32)
    ji_idx = ji_idx.astype(_i32)
    h, g = _stage_a(m_ji, e_rbf, W_mkj, b_mkj, W_e)
    hk, gj = _sc_gather(h, g, kj_idx, ji_idx)
    W_all = jnp.transpose(W_bil, (2, 1, 0)).reshape(EMBED, N_BIL * EMBED)
    W_all = W_all.astype(jnp.bfloat16)
    E_exp = jnp.kron(jnp.eye(N_BIL, dtype=_f32),
                     jnp.ones((1, EMBED), _f32)).astype(jnp.bfloat16)
    aggr = _stage_b(hk, gj, a_sbf, W_a, W_all, E_exp)
    return _sc_scatter(aggr, ji_idx)
